# baseline (speedup 1.0000x reference)
import functools

import jax
import jax.numpy as jnp
from jax import lax
from jax.experimental import pallas as pl
from jax.experimental.pallas import tpu as pltpu
from jax.experimental.pallas import tpu_sc as plsc

_D = 64
_B = 16384
_L = 16
_NC = 2
_NS = 16
_NW = _NC * _NS
_BPW = _B // _NW
_CHUNK = 128
_NCHUNK = _BPW // _CHUNK

_mesh = plsc.VectorSubcoreMesh(core_axis_name="c", subcore_axis_name="s")


@functools.partial(
    pl.kernel,
    mesh=_mesh,
    out_type=jax.ShapeDtypeStruct((_B,), jnp.float32),
    compiler_params=pltpu.CompilerParams(needs_layout_passes=False),
    scratch_types=[
        pltpu.VMEM((_BPW,), jnp.int32),
        pltpu.VMEM((_BPW,), jnp.int32),
        pltpu.VMEM((_BPW,), jnp.int32),
        pltpu.VMEM((2, _CHUNK, _D // 2), jnp.int32),
        pltpu.VMEM((2, _CHUNK, _D // 2), jnp.int32),
        pltpu.VMEM((2, _CHUNK, _D // 2), jnp.int32),
        pltpu.VMEM((_BPW,), jnp.float32),
        pltpu.SemaphoreType.DMA,
        pltpu.SemaphoreType.DMA,
        pltpu.SemaphoreType.DMA,
        pltpu.SemaphoreType.DMA,
        pltpu.SemaphoreType.DMA,
        pltpu.SemaphoreType.DMA,
        pltpu.SemaphoreType.DMA,
    ],
)
def _bt_norm_kernel(i_hbm, j_hbm, k_hbm, u_hbm, v_hbm, out_hbm,
                    ii_v, jj_v, kk_v, u_b, vj_b, vk_b, o_v,
                    s_idx, s_u0, s_vj0, s_vk0, s_u1, s_vj1, s_vk1):
    sem_sets = ((s_u0, s_vj0, s_vk0), (s_u1, s_vj1, s_vk1))
    wid = lax.axis_index("s") * _NC + lax.axis_index("c")
    base = wid * _BPW

    c1 = pltpu.async_copy(i_hbm.at[pl.ds(base, _BPW)], ii_v, s_idx)
    c2 = pltpu.async_copy(j_hbm.at[pl.ds(base, _BPW)], jj_v, s_idx)
    c3 = pltpu.async_copy(k_hbm.at[pl.ds(base, _BPW)], kk_v, s_idx)
    c1.wait()
    c2.wait()
    c3.wait()

    lane = lax.iota(jnp.int32, _L)
    hi_mask = lane == (_L - 1)

    def enqueue_chunk(c, buf):
        s_u, s_vj, s_vk = sem_sets[buf]

        def enq_body(g, _):
            row0 = c * _CHUNK + g * _L
            iv = ii_v[pl.ds(row0, _L)]
            jv = jj_v[pl.ds(row0, _L)]
            kv = kk_v[pl.ds(row0, _L)]
            for t in range(_L):
                r = g * _L + t
                pltpu.async_copy(
                    u_hbm.at[pl.ds(iv[t], 1)], u_b.at[buf, pl.ds(r, 1)], s_u)
                pltpu.async_copy(
                    v_hbm.at[pl.ds(jv[t], 1)], vj_b.at[buf, pl.ds(r, 1)], s_vj)
                pltpu.async_copy(
                    v_hbm.at[pl.ds(kv[t], 1)], vk_b.at[buf, pl.ds(r, 1)], s_vk)
            return 0

        lax.fori_loop(0, _CHUNK // _L, enq_body, 0, unroll=2)

    def drain_chunk(buf):
        s_u, s_vj, s_vk = sem_sets[buf]
        pltpu.make_async_copy(
            u_hbm.at[pl.ds(0, _CHUNK)], u_b.at[buf], s_u).wait()
        pltpu.make_async_copy(
            v_hbm.at[pl.ds(0, _CHUNK)], vj_b.at[buf], s_vj).wait()
        pltpu.make_async_copy(
            v_hbm.at[pl.ds(0, _CHUNK)], vk_b.at[buf], s_vk).wait()

    def compute_chunk(c, buf):
        def row_body(r, _):
            acc = jnp.zeros((_L,), jnp.float32)
            for q in range(_D // (2 * _L)):
                sl = pl.ds(q * _L, _L)
                ua, ub = plsc.unpack(
                    plsc.bitcast(u_b[buf, r, sl], jnp.bfloat16),
                    format=plsc.PackFormat.INTERLEAVED)
                ja, jb = plsc.unpack(
                    plsc.bitcast(vj_b[buf, r, sl], jnp.bfloat16),
                    format=plsc.PackFormat.INTERLEAVED)
                ka, kb = plsc.unpack(
                    plsc.bitcast(vk_b[buf, r, sl], jnp.bfloat16),
                    format=plsc.PackFormat.INTERLEAVED)
                dja = ua - ja
                dka = ua - ka
                djb = ub - jb
                dkb = ub - kb
                acc = acc + (dka * dka - dja * dja)
                acc = acc + (dkb * dkb - djb * djb)
            cum = plsc.cumsum(acc)
            sig = 1.0 / (1.0 + jnp.exp(-cum))
            pos = jnp.full((_L,), c * _CHUNK + r, jnp.int32)
            plsc.store_scatter(o_v, [pos], sig, mask=hi_mask)
            return 0

        lax.fori_loop(0, _CHUNK, row_body, 0, unroll=8)

    enqueue_chunk(0, 0)
    enqueue_chunk(1, 1)
    for c in range(_NCHUNK):
        buf = c % 2
        drain_chunk(buf)
        compute_chunk(c, buf)
        if c + 2 < _NCHUNK:
            enqueue_chunk(c + 2, buf)

    pltpu.sync_copy(o_v, out_hbm.at[pl.ds(base, _BPW)])


def kernel(i, j, k, u_weight, v_weight):
    def pack_bf16(w):
        wb = w.astype(jnp.bfloat16).reshape(w.shape[0], w.shape[1] // 2, 2)
        return jax.lax.bitcast_convert_type(wb, jnp.int32)

    return _bt_norm_kernel(
        i.astype(jnp.int32),
        j.astype(jnp.int32),
        k.astype(jnp.int32),
        pack_bf16(u_weight),
        pack_bf16(v_weight),
    )


# speedup vs baseline: 4.4951x; 4.4951x over previous
import functools

import jax
import jax.numpy as jnp
from jax import lax
from jax.experimental import pallas as pl
from jax.experimental.pallas import tpu as pltpu
from jax.experimental.pallas import tpu_sc as plsc

_D = 64
_B = 16384
_L = 16
_NC = 2
_NS = 16
_NW = _NC * _NS
_BPW = _B // _NW
_CHUNK = 128
_NCHUNK = _BPW // _CHUNK

_mesh = plsc.VectorSubcoreMesh(core_axis_name="c", subcore_axis_name="s")


@functools.partial(
    pl.kernel,
    mesh=_mesh,
    out_type=jax.ShapeDtypeStruct((_B,), jnp.float32),
    compiler_params=pltpu.CompilerParams(needs_layout_passes=False),
    scratch_types=[
        pltpu.VMEM((_BPW,), jnp.int32),
        pltpu.VMEM((_BPW,), jnp.int32),
        pltpu.VMEM((_BPW,), jnp.int32),
        pltpu.VMEM((2, _CHUNK, _D), jnp.float32),
        pltpu.VMEM((2, _CHUNK, _D), jnp.float32),
        pltpu.VMEM((2, _CHUNK, _D), jnp.float32),
        pltpu.VMEM((_BPW,), jnp.float32),
        pltpu.SemaphoreType.DMA,
        pltpu.SemaphoreType.DMA,
        pltpu.SemaphoreType.DMA,
        pltpu.SemaphoreType.DMA,
        pltpu.SemaphoreType.DMA,
        pltpu.SemaphoreType.DMA,
        pltpu.SemaphoreType.DMA,
    ],
)
def _bt_norm_kernel(i_hbm, j_hbm, k_hbm, u_hbm, v_hbm, out_hbm,
                    ii_v, jj_v, kk_v, u_b, vj_b, vk_b, o_v,
                    s_idx, s_u0, s_vj0, s_vk0, s_u1, s_vj1, s_vk1):
    sem_sets = ((s_u0, s_vj0, s_vk0), (s_u1, s_vj1, s_vk1))
    wid = lax.axis_index("s") * _NC + lax.axis_index("c")
    base = wid * _BPW

    c1 = pltpu.async_copy(i_hbm.at[pl.ds(base, _BPW)], ii_v, s_idx)
    c2 = pltpu.async_copy(j_hbm.at[pl.ds(base, _BPW)], jj_v, s_idx)
    c3 = pltpu.async_copy(k_hbm.at[pl.ds(base, _BPW)], kk_v, s_idx)
    c1.wait()
    c2.wait()
    c3.wait()

    lane = lax.iota(jnp.int32, _L)
    hi_mask = lane == (_L - 1)

    def enqueue_chunk(c, buf):
        s_u, s_vj, s_vk = sem_sets[buf]

        def enq_body(g, _):
            row0 = c * _CHUNK + g * _L
            iv = ii_v[pl.ds(row0, _L)]
            jv = jj_v[pl.ds(row0, _L)]
            kv = kk_v[pl.ds(row0, _L)]
            for t in range(_L):
                r = g * _L + t
                pltpu.async_copy(
                    u_hbm.at[pl.ds(iv[t], 1)], u_b.at[buf, pl.ds(r, 1)], s_u)
                pltpu.async_copy(
                    v_hbm.at[pl.ds(jv[t], 1)], vj_b.at[buf, pl.ds(r, 1)], s_vj)
                pltpu.async_copy(
                    v_hbm.at[pl.ds(kv[t], 1)], vk_b.at[buf, pl.ds(r, 1)], s_vk)
            return 0

        lax.fori_loop(0, _CHUNK // _L, enq_body, 0, unroll=2)

    def drain_chunk(buf):
        s_u, s_vj, s_vk = sem_sets[buf]
        pltpu.make_async_copy(
            u_hbm.at[pl.ds(0, _CHUNK)], u_b.at[buf], s_u).wait()
        pltpu.make_async_copy(
            v_hbm.at[pl.ds(0, _CHUNK)], vj_b.at[buf], s_vj).wait()
        pltpu.make_async_copy(
            v_hbm.at[pl.ds(0, _CHUNK)], vk_b.at[buf], s_vk).wait()

    def compute_chunk(c, buf):
        def row_body(r, _):
            acc = jnp.zeros((_L,), jnp.float32)
            for q in range(_D // _L):
                sl = pl.ds(q * _L, _L)
                u16 = u_b[buf, r, sl]
                dj = u16 - vj_b[buf, r, sl]
                dk = u16 - vk_b[buf, r, sl]
                acc = acc + (dk * dk - dj * dj)
            cum = plsc.cumsum(acc)
            sig = 1.0 / (1.0 + jnp.exp(-cum))
            pos = jnp.full((_L,), c * _CHUNK + r, jnp.int32)
            plsc.store_scatter(o_v, [pos], sig, mask=hi_mask)
            return 0

        lax.fori_loop(0, _CHUNK, row_body, 0, unroll=8)

    enqueue_chunk(0, 0)
    enqueue_chunk(1, 1)
    for c in range(_NCHUNK):
        buf = c % 2
        drain_chunk(buf)
        compute_chunk(c, buf)
        if c + 2 < _NCHUNK:
            enqueue_chunk(c + 2, buf)

    pltpu.sync_copy(o_v, out_hbm.at[pl.ds(base, _BPW)])


def kernel(i, j, k, u_weight, v_weight):
    return _bt_norm_kernel(
        i.astype(jnp.int32),
        j.astype(jnp.int32),
        k.astype(jnp.int32),
        u_weight,
        v_weight,
    )
